# baseline (speedup 1.0000x reference)
import functools

import jax
import jax.numpy as jnp
from jax import lax
from jax.experimental import pallas as pl
from jax.experimental.pallas import tpu as pltpu
from jax.experimental.pallas import tpu_sc as plsc

_NUM_CENTROIDS = 1024
_EMA_DECAY = 0.99
_GAMMA = 1.0
_BN = 1024
_BN2 = 2048
_NCAND = 4
_LN2 = 0.6931471805599453
_NC, _NS = 2, 16
_NW = _NC * _NS


def _sim_block(x_ref, cb_ref, cands_ref, psum_ref, hclust_ref, ql_ref):
    i = pl.program_id(0)
    nblocks = pl.num_programs(0)

    @pl.when(i == 0)
    def _init():
        psum_ref[...] = jnp.zeros_like(psum_ref)
        hclust_ref[...] = jnp.zeros_like(hclust_ref)

    x = x_ref[...]
    cb = cb_ref[...]
    sim = jax.lax.dot_general(x, cb, (((1,), (1,)), ((), ())),
                              preferred_element_type=jnp.float32)
    iota = jax.lax.broadcasted_iota(jnp.int32, sim.shape, 1)

    e = jnp.exp(sim)
    denom = jnp.sum(e, axis=-1, keepdims=True)
    psum_ref[...] += jnp.sum(e / denom, axis=0, keepdims=True)
    ent_row = jnp.sum(e * sim, axis=-1, keepdims=True)
    hc_rows = ent_row / (denom * _LN2) - jnp.log2(denom)
    hclust_ref[...] += jnp.sum(hc_rows).reshape(1, 1)

    work = sim
    cands = []
    for _ in range(_NCAND):
        c = jnp.argmax(work, axis=-1, keepdims=True).astype(jnp.int32)
        cands.append(c)
        work = jnp.where(iota == c, -jnp.inf, work)
    cands_ref[...] = jnp.concatenate(cands, axis=1)

    @pl.when(i == nblocks - 1)
    def _finish():
        n = nblocks * _BN
        h_clust = -(hclust_ref[...] / n)
        div = psum_ref[...] / n
        h_div = -jnp.sum(div * jnp.log2(div + 1e-8)).reshape(1, 1)
        ql_ref[...] = h_clust - _GAMMA * h_div


def _run_sim(flat, codebook):
    n, d = flat.shape
    k = codebook.shape[0]
    nblocks = n // _BN
    out_shapes = (
        jax.ShapeDtypeStruct((n, _NCAND), jnp.int32),
        jax.ShapeDtypeStruct((1, k), jnp.float32),
        jax.ShapeDtypeStruct((1, 1), jnp.float32),
        jax.ShapeDtypeStruct((1, 1), jnp.float32),
    )
    in_specs = [pl.BlockSpec((_BN, d), lambda i: (i, 0)),
                pl.BlockSpec((k, d), lambda i: (0, 0))]
    out_specs = (
        pl.BlockSpec((_BN, _NCAND), lambda i: (i, 0)),
        pl.BlockSpec((1, k), lambda i: (0, 0)),
        pl.BlockSpec((1, 1), lambda i: (0, 0)),
        pl.BlockSpec((1, 1), lambda i: (0, 0)),
    )
    return pl.pallas_call(
        _sim_block,
        grid=(nblocks,),
        in_specs=in_specs,
        out_specs=out_specs,
        out_shape=out_shapes,
        compiler_params=pltpu.CompilerParams(
            dimension_semantics=("arbitrary",)),
    )(flat, codebook)


def _sc_gather(table, idx):
    b = idx.shape[0]
    d = table.shape[1]
    b_per_w = b // _NW
    chunk = 512
    nchunks = b_per_w // chunk
    mesh = plsc.VectorSubcoreMesh(core_axis_name="c", subcore_axis_name="s")

    @functools.partial(
        pl.kernel, mesh=mesh,
        out_type=jax.ShapeDtypeStruct((b, d), jnp.float32),
        scratch_types=[
            pltpu.VMEM((chunk,), jnp.int32),
            pltpu.VMEM((chunk, d), jnp.float32),
            pltpu.SemaphoreType.DMA,
        ],
    )
    def _gather_kernel(table_hbm, idx_hbm, out_hbm, idx_v, rows_v, sem):
        wid = lax.axis_index("s") * _NC + lax.axis_index("c")
        base = wid * b_per_w
        for j in range(nchunks):
            off = base + j * chunk
            pltpu.sync_copy(idx_hbm.at[pl.ds(off, chunk)], idx_v)
            pltpu.async_copy(table_hbm.at[idx_v], rows_v, sem).wait()
            pltpu.sync_copy(rows_v, out_hbm.at[pl.ds(off, chunk)])

    return _gather_kernel(table, idx)


def _pick_block(x_ref, rows_ref, cands_ref, quant_ref, idx_ref, counts_ref):
    i = pl.program_id(0)

    @pl.when(i == 0)
    def _init():
        counts_ref[...] = jnp.zeros_like(counts_ref)

    x = x_ref[...]
    d = x.shape[1]
    rows4 = rows_ref[...]
    call = cands_ref[...]
    k = counts_ref.shape[1]

    cands, dots, rows = [], [], []
    for t in range(_NCAND):
        row = rows4[:, t * 128:t * 128 + d]
        rows.append(row)
        cands.append(call[:, t:t + 1])
        dots.append(jnp.sum(x * row, axis=-1, keepdims=True))

    best = dots[0]
    for d_ in dots[1:]:
        best = jnp.maximum(best, d_)
    idx = jnp.full_like(cands[0], k)
    for c, d_ in zip(cands, dots):
        idx = jnp.where(d_ == best, jnp.minimum(idx, c), idx)
    idx_ref[...] = idx

    iota = jax.lax.broadcasted_iota(jnp.int32, (x.shape[0], k), 1)
    counts_ref[...] += jnp.sum((iota == idx).astype(jnp.float32),
                               axis=0, keepdims=True)

    q = rows[0]
    for c, row in zip(cands[1:], rows[1:]):
        q = jnp.where(c == idx, row, q)
    qc = q - jnp.mean(q, axis=-1, keepdims=True)
    quant_ref[...] = qc / jnp.sqrt(jnp.sum(qc * qc, axis=-1, keepdims=True))


def _run_pick(flat, rows4, cands):
    n, d = flat.shape
    k = _NUM_CENTROIDS
    nblocks = n // _BN2
    out_shapes = (
        jax.ShapeDtypeStruct((n, d), jnp.float32),
        jax.ShapeDtypeStruct((n, 1), jnp.int32),
        jax.ShapeDtypeStruct((1, k), jnp.float32),
    )
    in_specs = [
        pl.BlockSpec((_BN2, d), lambda i: (i, 0)),
        pl.BlockSpec((_BN2, _NCAND * 128), lambda i: (i, 0)),
        pl.BlockSpec((_BN2, _NCAND), lambda i: (i, 0)),
    ]
    out_specs = (
        pl.BlockSpec((_BN2, d), lambda i: (i, 0)),
        pl.BlockSpec((_BN2, 1), lambda i: (i, 0)),
        pl.BlockSpec((1, k), lambda i: (0, 0)),
    )
    return pl.pallas_call(
        _pick_block,
        grid=(nblocks,),
        in_specs=in_specs,
        out_specs=out_specs,
        out_shape=out_shapes,
        compiler_params=pltpu.CompilerParams(
            dimension_semantics=("arbitrary",)),
    )(flat, rows4, cands)


def kernel(inputs, codebook, cluster_counts, train):
    b, t, d = inputs.shape
    n = b * t
    flat = inputs.reshape(n, d)
    cands, _psum, _hc, ql = _run_sim(flat, codebook)
    idxflat = cands.reshape(n * _NCAND)
    cbpad = jnp.pad(codebook, ((0, 0), (0, 128 - d)))
    gathered = _sc_gather(cbpad, idxflat)
    rows4 = gathered.reshape(n, _NCAND * 128)
    quant, idx, counts = _run_pick(flat, rows4, cands)
    quantized = quant.reshape(inputs.shape)
    nn_idx = idx.reshape(1, b, t)
    quantization_loss = jnp.broadcast_to(ql[0, 0], (b, t, _NUM_CENTROIDS))
    counts_out = jnp.where(train,
                           _EMA_DECAY * cluster_counts
                           + (1.0 - _EMA_DECAY) * counts[0],
                           cluster_counts)
    codebook_values = codebook[None]
    return (quantized, quantization_loss, nn_idx, codebook_values, counts_out)


# speedup vs baseline: 1.2085x; 1.2085x over previous
import jax
import jax.numpy as jnp
from jax.experimental import pallas as pl
from jax.experimental.pallas import tpu as pltpu

_NUM_CENTROIDS = 1024
_EMA_DECAY = 0.99
_GAMMA = 1.0
_BN = 1024
_NCAND = 4
_LN2 = 0.6931471805599453


def _vq_block(x_ref, cb_ref, cbh_ref, cbm_ref, cbl_ref,
              quant_ref, idx_ref, counts_ref, psum_ref, hclust_ref, ql_ref):
    i = pl.program_id(0)
    nblocks = pl.num_programs(0)

    @pl.when(i == 0)
    def _init():
        counts_ref[...] = jnp.zeros_like(counts_ref)
        psum_ref[...] = jnp.zeros_like(psum_ref)
        hclust_ref[...] = jnp.zeros_like(hclust_ref)

    x = x_ref[...]
    cb = cb_ref[...]
    k = cb.shape[0]
    sim = jax.lax.dot_general(x, cb, (((1,), (1,)), ((), ())),
                              preferred_element_type=jnp.float32)
    iota = jax.lax.broadcasted_iota(jnp.int32, sim.shape, 1)

    e = jnp.exp(sim)
    denom = jnp.sum(e, axis=-1, keepdims=True)
    psum_ref[...] += jnp.sum(e / denom, axis=0, keepdims=True)
    ent_row = jnp.sum(e * sim, axis=-1, keepdims=True)
    hc_rows = ent_row / (denom * _LN2) - jnp.log2(denom)
    hclust_ref[...] += jnp.sum(hc_rows).reshape(1, 1)

    cbh, cbm, cbl = cbh_ref[...], cbm_ref[...], cbl_ref[...]
    work = sim
    cands, dots, rows = [], [], []
    for _ in range(_NCAND):
        c = jnp.argmax(work, axis=-1, keepdims=True).astype(jnp.int32)
        onehot = (iota == c).astype(jnp.float32)
        work = jnp.where(onehot != 0.0, -jnp.inf, work)
        row = (jax.lax.dot_general(onehot, cbh, (((1,), (0,)), ((), ())),
                                   preferred_element_type=jnp.float32)
               + jax.lax.dot_general(onehot, cbm, (((1,), (0,)), ((), ())),
                                     preferred_element_type=jnp.float32)
               + jax.lax.dot_general(onehot, cbl, (((1,), (0,)), ((), ())),
                                     preferred_element_type=jnp.float32))
        cands.append(c)
        rows.append(row)
        dots.append(jnp.sum(x * row, axis=-1, keepdims=True))

    best = dots[0]
    for d_ in dots[1:]:
        best = jnp.maximum(best, d_)
    idx = jnp.full_like(cands[0], k)
    for c, d_ in zip(cands, dots):
        idx = jnp.where(d_ == best, jnp.minimum(idx, c), idx)
    idx_ref[...] = idx

    counts_ref[...] += jnp.sum((iota == idx).astype(jnp.float32),
                               axis=0, keepdims=True)

    q = rows[0]
    for c, row in zip(cands[1:], rows[1:]):
        q = jnp.where(c == idx, row, q)
    qc = q - jnp.mean(q, axis=-1, keepdims=True)
    quant_ref[...] = qc / jnp.sqrt(jnp.sum(qc * qc, axis=-1, keepdims=True))

    @pl.when(i == nblocks - 1)
    def _finish():
        n = nblocks * _BN
        h_clust = -(hclust_ref[...] / n)
        div = psum_ref[...] / n
        h_div = -jnp.sum(div * jnp.log2(div + 1e-8)).reshape(1, 1)
        ql_ref[...] = h_clust - _GAMMA * h_div


def _split_bf16x3(cb):
    bits = cb.view(jnp.int32)
    hi = jnp.bitwise_and(bits, jnp.int32(-65536)).view(jnp.float32)
    rem = cb - hi
    rbits = rem.view(jnp.int32)
    mid = jnp.bitwise_and(rbits, jnp.int32(-65536)).view(jnp.float32)
    lo = rem - mid
    return hi, mid, lo


def _run_vq(flat, codebook):
    n, d = flat.shape
    k = codebook.shape[0]
    nblocks = n // _BN
    cbh, cbm, cbl = _split_bf16x3(codebook)
    out_shapes = (
        jax.ShapeDtypeStruct((n, d), jnp.float32),
        jax.ShapeDtypeStruct((n, 1), jnp.int32),
        jax.ShapeDtypeStruct((1, k), jnp.float32),
        jax.ShapeDtypeStruct((1, k), jnp.float32),
        jax.ShapeDtypeStruct((1, 1), jnp.float32),
        jax.ShapeDtypeStruct((1, 1), jnp.float32),
    )
    cb_spec = pl.BlockSpec((k, d), lambda i: (0, 0))
    in_specs = [pl.BlockSpec((_BN, d), lambda i: (i, 0)),
                cb_spec, cb_spec, cb_spec, cb_spec]
    out_specs = (
        pl.BlockSpec((_BN, d), lambda i: (i, 0)),
        pl.BlockSpec((_BN, 1), lambda i: (i, 0)),
        pl.BlockSpec((1, k), lambda i: (0, 0)),
        pl.BlockSpec((1, k), lambda i: (0, 0)),
        pl.BlockSpec((1, 1), lambda i: (0, 0)),
        pl.BlockSpec((1, 1), lambda i: (0, 0)),
    )
    return pl.pallas_call(
        _vq_block,
        grid=(nblocks,),
        in_specs=in_specs,
        out_specs=out_specs,
        out_shape=out_shapes,
        compiler_params=pltpu.CompilerParams(
            dimension_semantics=("arbitrary",)),
    )(flat, codebook, cbh, cbm, cbl)


def kernel(inputs, codebook, cluster_counts, train):
    b, t, d = inputs.shape
    n = b * t
    flat = inputs.reshape(n, d)
    quant, idx, counts, _psum, _hc, ql = _run_vq(flat, codebook)
    quantized = quant.reshape(inputs.shape)
    nn_idx = idx.reshape(1, b, t)
    quantization_loss = jnp.broadcast_to(ql[0, 0], (b, t, _NUM_CENTROIDS))
    counts_out = jnp.where(train,
                           _EMA_DECAY * cluster_counts
                           + (1.0 - _EMA_DECAY) * counts[0],
                           cluster_counts)
    codebook_values = codebook[None]
    return (quantized, quantization_loss, nn_idx, codebook_values, counts_out)


# speedup vs baseline: 1.2285x; 1.0166x over previous
import jax
import jax.numpy as jnp
from jax.experimental import pallas as pl
from jax.experimental.pallas import tpu as pltpu

_NUM_CENTROIDS = 1024
_EMA_DECAY = 0.99
_GAMMA = 1.0
_BN = 1024
_NCAND = 4
_LN2 = 0.6931471805599453


def _vq_block(x_ref, cb_ref, cbh_ref, cbm_ref, cbl_ref,
              quant_ref, idx_ref, counts_ref, psum_ref, hclust_ref, ql_ref):
    i = pl.program_id(0)
    nblocks = pl.num_programs(0)

    @pl.when(i == 0)
    def _init():
        counts_ref[...] = jnp.zeros_like(counts_ref)
        psum_ref[...] = jnp.zeros_like(psum_ref)
        hclust_ref[...] = jnp.zeros_like(hclust_ref)

    x = x_ref[...]
    cb = cb_ref[...]
    k = cb.shape[0]
    sim = jax.lax.dot_general(x, cb, (((1,), (1,)), ((), ())),
                              preferred_element_type=jnp.float32)
    iota = jax.lax.broadcasted_iota(jnp.int32, sim.shape, 1)

    e = jnp.exp(sim)
    denom = jnp.sum(e, axis=-1, keepdims=True)
    psum_ref[...] += jnp.sum(e / denom, axis=0, keepdims=True)
    ent_row = jnp.sum(e * sim, axis=-1, keepdims=True)
    hc_rows = ent_row / (denom * _LN2) - jnp.log2(denom)
    hclust_ref[...] += jnp.sum(hc_rows).reshape(1, 1)

    cbh, cbm, cbl = cbh_ref[...], cbm_ref[...], cbl_ref[...]
    work = sim
    cands, dots, rows = [], [], []
    for _ in range(_NCAND):
        c = jnp.argmax(work, axis=-1, keepdims=True).astype(jnp.int32)
        onehot = (iota == c).astype(jnp.float32)
        work = jnp.where(iota == c, -jnp.inf, work)
        row = (jax.lax.dot_general(onehot, cbh, (((1,), (0,)), ((), ())),
                                   preferred_element_type=jnp.float32)
               + jax.lax.dot_general(onehot, cbm, (((1,), (0,)), ((), ())),
                                     preferred_element_type=jnp.float32)
               + jax.lax.dot_general(onehot, cbl, (((1,), (0,)), ((), ())),
                                     preferred_element_type=jnp.float32))
        cands.append(c)
        rows.append(row)
        dots.append(jnp.sum(x * row, axis=-1, keepdims=True))

    best = dots[0]
    for d_ in dots[1:]:
        best = jnp.maximum(best, d_)
    idx = jnp.full_like(cands[0], k)
    for c, d_ in zip(cands, dots):
        idx = jnp.where(d_ == best, jnp.minimum(idx, c), idx)
    idx_ref[...] = idx

    counts_ref[...] += jnp.sum((iota == idx).astype(jnp.float32),
                               axis=0, keepdims=True)

    q = rows[0]
    for c, row in zip(cands[1:], rows[1:]):
        q = jnp.where(c == idx, row, q)
    qc = q - jnp.mean(q, axis=-1, keepdims=True)
    quant_ref[...] = qc / jnp.sqrt(jnp.sum(qc * qc, axis=-1, keepdims=True))

    @pl.when(i == nblocks - 1)
    def _finish():
        n = nblocks * _BN
        h_clust = -(hclust_ref[...] / n)
        div = psum_ref[...] / n
        h_div = -jnp.sum(div * jnp.log2(div + 1e-8)).reshape(1, 1)
        ql_ref[...] = h_clust - _GAMMA * h_div


def _split_bf16x3(cb):
    bits = cb.view(jnp.int32)
    hi = jnp.bitwise_and(bits, jnp.int32(-65536)).view(jnp.float32)
    rem = cb - hi
    rbits = rem.view(jnp.int32)
    mid = jnp.bitwise_and(rbits, jnp.int32(-65536)).view(jnp.float32)
    lo = rem - mid
    return hi, mid, lo


def _run_vq(flat, codebook):
    n, d = flat.shape
    k = codebook.shape[0]
    nblocks = n // _BN
    cbh, cbm, cbl = _split_bf16x3(codebook)
    out_shapes = (
        jax.ShapeDtypeStruct((n, d), jnp.float32),
        jax.ShapeDtypeStruct((n, 1), jnp.int32),
        jax.ShapeDtypeStruct((1, k), jnp.float32),
        jax.ShapeDtypeStruct((1, k), jnp.float32),
        jax.ShapeDtypeStruct((1, 1), jnp.float32),
        jax.ShapeDtypeStruct((1, 1), jnp.float32),
    )
    cb_spec = pl.BlockSpec((k, d), lambda i: (0, 0))
    in_specs = [pl.BlockSpec((_BN, d), lambda i: (i, 0)),
                cb_spec, cb_spec, cb_spec, cb_spec]
    out_specs = (
        pl.BlockSpec((_BN, d), lambda i: (i, 0)),
        pl.BlockSpec((_BN, 1), lambda i: (i, 0)),
        pl.BlockSpec((1, k), lambda i: (0, 0)),
        pl.BlockSpec((1, k), lambda i: (0, 0)),
        pl.BlockSpec((1, 1), lambda i: (0, 0)),
        pl.BlockSpec((1, 1), lambda i: (0, 0)),
    )
    return pl.pallas_call(
        _vq_block,
        grid=(nblocks,),
        in_specs=in_specs,
        out_specs=out_specs,
        out_shape=out_shapes,
        compiler_params=pltpu.CompilerParams(
            dimension_semantics=("arbitrary",)),
    )(flat, codebook, cbh, cbm, cbl)


def kernel(inputs, codebook, cluster_counts, train):
    b, t, d = inputs.shape
    n = b * t
    flat = inputs.reshape(n, d)
    quant, idx, counts, _psum, _hc, ql = _run_vq(flat, codebook)
    quantized = quant.reshape(inputs.shape)
    nn_idx = idx.reshape(1, b, t)
    quantization_loss = jnp.broadcast_to(ql[0, 0], (b, t, _NUM_CENTROIDS))
    counts_out = jnp.where(train,
                           _EMA_DECAY * cluster_counts
                           + (1.0 - _EMA_DECAY) * counts[0],
                           cluster_counts)
    codebook_values = codebook[None]
    return (quantized, quantization_loss, nn_idx, codebook_values, counts_out)
